# Initial kernel scaffold; baseline (speedup 1.0000x reference)
import jax
import jax.numpy as jnp
from jax.experimental import pallas as pl


def kernel(node_features, edge_index, W1, b1, W2, b2):
    raise NotImplementedError("write your pallas kernel here")


# speedup vs baseline: 6.7268x; 6.7268x over previous
import dataclasses
import functools

import jax
import jax.numpy as jnp
from jax import lax
from jax.experimental import pallas as pl
from jax.experimental.pallas import tpu as pltpu
from jax.experimental.pallas import tpu_sc as plsc

N = 10000
E = 320000
D = 128

NC, NS = 2, 16
NW = NC * NS
EPW = E // NW
CHUNK = 128
FULL = EPW // CHUNK
REM = EPW - FULL * CHUNK

ZROWS = 200
NZCH = N // ZROWS
ZCH = N // 16


def _sc_agg_body(with_deg, table, src, dst, znd, *rest):
    if with_deg:
        outf, outd, sidx, didx, rows, sidx_r, didx_r, rows_r, acc, dacc, sem = rest
    else:
        outf, sidx, didx, rows, sidx_r, didx_r, rows_r, acc, sem = rest

    c = lax.axis_index("c")
    s = lax.axis_index("s")
    w = s * NC + c

    for j in range(4):
        i = s + NS * j

        @pl.when(i < NZCH)
        def _():
            r0 = i * ZROWS
            pltpu.sync_copy(znd.at[pl.ds(r0, ZROWS)], acc.at[pl.ds(r0, ZROWS)])

    if with_deg:
        z16 = jnp.zeros((16,), jnp.float32)

        @pl.loop(0, ZCH)
        def _(i):
            dacc[pl.ds(i * 16, 16)] = z16

    plsc.subcore_barrier()

    ebase = w * EPW
    ones16 = jnp.full((16,), 1.0, jnp.float32)

    @pl.loop(0, FULL)
    def _(i):
        base = ebase + i * CHUNK
        pltpu.sync_copy(src.at[pl.ds(base, CHUNK)], sidx)
        pltpu.sync_copy(dst.at[pl.ds(base, CHUNK)], didx)
        pltpu.async_copy(table.at[sidx], rows, sem).wait()
        pltpu.sync_copy(rows, acc.at[didx], add=True)
        if with_deg:
            for g in range(CHUNK // 16):
                plsc.addupdate_scatter(dacc, [didx[pl.ds(g * 16, 16)]], ones16)

    rbase = ebase + FULL * CHUNK
    pltpu.sync_copy(src.at[pl.ds(rbase, REM)], sidx_r)
    pltpu.sync_copy(dst.at[pl.ds(rbase, REM)], didx_r)
    pltpu.async_copy(table.at[sidx_r], rows_r, sem).wait()
    pltpu.sync_copy(rows_r, acc.at[didx_r], add=True)
    if with_deg:
        for g in range(REM // 16):
            plsc.addupdate_scatter(dacc, [didx_r[pl.ds(g * 16, 16)]], ones16)
        pltpu.sync_copy(dacc, outd.at[pl.ds(w * N, N)])

    plsc.subcore_barrier()

    for j in range(4):
        i = s + NS * j

        @pl.when(i < NZCH)
        def _():
            r0 = i * ZROWS
            pltpu.sync_copy(acc.at[pl.ds(r0, ZROWS)],
                            outf.at[pl.ds(c * N + r0, ZROWS)])


def _make_sc_agg(with_deg):
    mesh = plsc.VectorSubcoreMesh(core_axis_name="c", subcore_axis_name="s")
    out_type = jax.ShapeDtypeStruct((NC * N, D), jnp.float32)
    if with_deg:
        out_type = (out_type, jax.ShapeDtypeStruct((NW * N,), jnp.float32))
    scratch = [
        pltpu.VMEM((CHUNK,), jnp.int32),
        pltpu.VMEM((CHUNK,), jnp.int32),
        pltpu.VMEM((CHUNK, D), jnp.float32),
        pltpu.VMEM((REM,), jnp.int32),
        pltpu.VMEM((REM,), jnp.int32),
        pltpu.VMEM((REM, D), jnp.float32),
        pltpu.VMEM_SHARED((N, D), jnp.float32),
    ]
    if with_deg:
        scratch.append(pltpu.VMEM((N,), jnp.float32))
    scratch.append(pltpu.SemaphoreType.DMA)
    cp = None
    if with_deg:
        cp = pltpu.CompilerParams()
        if "needs_layout_passes" in pltpu.CompilerParams.__dataclass_fields__:
            cp = dataclasses.replace(cp, needs_layout_passes=False)
    return pl.kernel(functools.partial(_sc_agg_body, with_deg),
                     out_type=out_type, mesh=mesh, scratch_types=scratch,
                     compiler_params=cp)


_sc_agg_deg = _make_sc_agg(True)
_sc_agg = _make_sc_agg(False)


BN = 400


def _linear_body(relu, p0, p1, dt, w, b, o):
    deg = jnp.maximum(jnp.sum(dt[...], axis=1, keepdims=True), 1.0)
    mean = (p0[...] + p1[...]) / deg
    out = lax.dot_general(mean, w[...], (((1,), (1,)), ((), ())),
                          preferred_element_type=jnp.float32)
    out = out + b[...]
    if relu:
        out = jnp.maximum(out, 0.0)
    o[...] = out


def _linear(featp, degt, W, b, relu):
    nb = N // BN
    return pl.pallas_call(
        functools.partial(_linear_body, relu),
        grid=(nb,),
        in_specs=[
            pl.BlockSpec((BN, D), lambda i: (i, 0)),
            pl.BlockSpec((BN, D), lambda i: (i + nb, 0)),
            pl.BlockSpec((BN, NW), lambda i: (i, 0)),
            pl.BlockSpec((D, D), lambda i: (0, 0)),
            pl.BlockSpec((1, D), lambda i: (0, 0)),
        ],
        out_specs=pl.BlockSpec((BN, D), lambda i: (i, 0)),
        out_shape=jax.ShapeDtypeStruct((N, D), jnp.float32),
    )(featp, featp, degt, W, b.reshape(1, D))


def kernel(node_features, edge_index, W1, b1, W2, b2):
    src = edge_index[0].astype(jnp.int32)
    dst = edge_index[1].astype(jnp.int32)
    znd = jnp.zeros((N, D), jnp.float32)

    featp1, degp = _sc_agg_deg(node_features, src, dst, znd)
    degt = degp.reshape(NW, N).T
    h1 = _linear(featp1, degt, W1, b1, relu=True)
    featp2 = _sc_agg(h1, src, dst, znd)
    return _linear(featp2, degt, W2, b2, relu=False)


# speedup vs baseline: 11.1223x; 1.6534x over previous
import dataclasses
import functools

import jax
import jax.numpy as jnp
from jax import lax
from jax.experimental import pallas as pl
from jax.experimental.pallas import tpu as pltpu
from jax.experimental.pallas import tpu_sc as plsc

N = 10000
E = 320000
D = 128

NC, NS = 2, 16
NW = NC * NS
CHUNK = 128
NCH = E // CHUNK
BCH = NCH // NW
XCH = NCH - BCH * NW

ZROWS = 200
NZCH = N // ZROWS
ZCH = N // 16


IBLK = 13
NBLK = BCH // IBLK


def _sc_agg_body(with_deg, table, src2d, dst2d, znd, *rest):
    if with_deg:
        (outf, outd, sidx0, sidx1, didx0, didx1, rows0, rows1, acc, dacc,
         g0, g1, s0, s1, i0, i1) = rest
    else:
        (outf, sidx0, sidx1, didx0, didx1, rows0, rows1, acc,
         g0, g1, s0, s1, i0, i1) = rest

    c = lax.axis_index("c")
    s = lax.axis_index("s")
    w = s * NC + c

    for j in range(4):
        i = s + NS * j

        @pl.when(i < NZCH)
        def _():
            r0 = i * ZROWS
            pltpu.sync_copy(znd.at[pl.ds(r0, ZROWS)], acc.at[pl.ds(r0, ZROWS)])

    cb = w * BCH
    sidxb = (sidx0, sidx1)
    didxb = (didx0, didx1)
    isem = (i0, i1)
    rowsb = (rows0, rows1)
    gsem = (g0, g1)
    ssem = (s0, s1)
    ones16 = jnp.full((16,), 1.0, jnp.float32)

    pltpu.sync_copy(src2d.at[pl.ds(cb, IBLK)], sidx0)
    pltpu.sync_copy(dst2d.at[pl.ds(cb, IBLK)], didx0)

    if with_deg:
        z16 = jnp.zeros((16,), jnp.float32)

        @pl.loop(0, ZCH)
        def _(i):
            dacc[pl.ds(i * 16, 16)] = z16

    plsc.subcore_barrier()

    def issue_idx_block(m):
        q = m % 2
        pltpu.async_copy(src2d.at[pl.ds(cb + m * IBLK, IBLK)], sidxb[q],
                         isem[q])
        pltpu.async_copy(dst2d.at[pl.ds(cb + m * IBLK, IBLK)], didxb[q],
                         isem[q])

    def wait_idx_block(m):
        q = m % 2
        pltpu.make_async_copy(src2d.at[pl.ds(cb + m * IBLK, IBLK)], sidxb[q],
                              isem[q]).wait()
        pltpu.make_async_copy(dst2d.at[pl.ds(cb + m * IBLK, IBLK)], didxb[q],
                              isem[q]).wait()

    def sidx_row(k):
        m, j = divmod(k, IBLK)
        return sidxb[m % 2].at[0] if k == BCH else sidxb[m % 2].at[j]

    def didx_row(k):
        m, j = divmod(k, IBLK)
        return didxb[m % 2].at[0] if k == BCH else didxb[m % 2].at[j]

    def issue_gather(k):
        pltpu.async_copy(table.at[sidx_row(k)], rowsb[k % 2], gsem[k % 2])

    def wait_gather(k):
        pltpu.make_async_copy(table.at[sidx_row(k)], rowsb[k % 2],
                              gsem[k % 2]).wait()

    def start_scatter(k):
        pltpu.async_copy(rowsb[k % 2], acc.at[didx_row(k)], ssem[k % 2],
                         add=True)

    def wait_scatter(k):
        pltpu.make_async_copy(rowsb[k % 2], acc.at[didx_row(k)],
                              ssem[k % 2]).wait()

    def deg_update(k):
        if with_deg:
            m, j = divmod(k, IBLK)
            j = 0 if k == BCH else j
            for g in range(CHUNK // 16):
                plsc.addupdate_scatter(
                    dacc, [didxb[m % 2][j, pl.ds(g * 16, 16)]], ones16)

    for k in range(BCH):
        m, j = divmod(k, IBLK)
        if k == 0:
            issue_gather(0)
        wait_gather(k)
        start_scatter(k)
        if k > 0:
            wait_scatter(k - 1)
        if k < BCH - 1:
            if j == IBLK - 1:
                wait_idx_block(m + 1)
            issue_gather(k + 1)
        else:
            @pl.when(w < XCH)
            def _():
                xb = NW * BCH + w
                pltpu.make_async_copy(src2d.at[pl.ds(xb, 1)],
                                      sidxb[0].at[pl.ds(0, 1)],
                                      isem[0]).wait()
                pltpu.make_async_copy(dst2d.at[pl.ds(xb, 1)],
                                      didxb[0].at[pl.ds(0, 1)],
                                      isem[0]).wait()
                issue_gather(BCH)

        deg_update(k)
        if j == 1:
            if m + 1 < NBLK:
                issue_idx_block(m + 1)
            else:
                @pl.when(w < XCH)
                def _():
                    xb = NW * BCH + w
                    q = (m + 1) % 2
                    pltpu.async_copy(src2d.at[pl.ds(xb, 1)],
                                     sidxb[q].at[pl.ds(0, 1)], isem[q])
                    pltpu.async_copy(dst2d.at[pl.ds(xb, 1)],
                                     didxb[q].at[pl.ds(0, 1)], isem[q])

    @pl.when(w < XCH)
    def _():
        wait_gather(BCH)
        start_scatter(BCH)
        wait_scatter(BCH - 1)
        deg_update(BCH)
        wait_scatter(BCH)

    @pl.when(w >= XCH)
    def _():
        wait_scatter(BCH - 1)

    if with_deg:
        pltpu.sync_copy(dacc, outd.at[pl.ds(w * N, N)])

    plsc.subcore_barrier()

    for j in range(4):
        i = s + NS * j

        @pl.when(i < NZCH)
        def _():
            r0 = i * ZROWS
            pltpu.sync_copy(acc.at[pl.ds(r0, ZROWS)],
                            outf.at[pl.ds(c * N + r0, ZROWS)])


def _make_sc_agg(with_deg):
    mesh = plsc.VectorSubcoreMesh(core_axis_name="c", subcore_axis_name="s")
    out_type = jax.ShapeDtypeStruct((NC * N, D), jnp.float32)
    if with_deg:
        out_type = (out_type, jax.ShapeDtypeStruct((NW * N,), jnp.float32))
    scratch = [
        pltpu.VMEM((IBLK, CHUNK), jnp.int32),
        pltpu.VMEM((IBLK, CHUNK), jnp.int32),
        pltpu.VMEM((IBLK, CHUNK), jnp.int32),
        pltpu.VMEM((IBLK, CHUNK), jnp.int32),
        pltpu.VMEM((CHUNK, D), jnp.float32),
        pltpu.VMEM((CHUNK, D), jnp.float32),
        pltpu.VMEM_SHARED((N, D), jnp.float32),
    ]
    if with_deg:
        scratch.append(pltpu.VMEM((N,), jnp.float32))
    scratch += [pltpu.SemaphoreType.DMA] * 6
    cp = pltpu.CompilerParams()
    fields = pltpu.CompilerParams.__dataclass_fields__
    if "use_tc_tiling_on_sc" in fields:
        cp = dataclasses.replace(cp, use_tc_tiling_on_sc=False)
    if with_deg and "needs_layout_passes" in fields:
        cp = dataclasses.replace(cp, needs_layout_passes=False)
    return pl.kernel(functools.partial(_sc_agg_body, with_deg),
                     out_type=out_type, mesh=mesh, scratch_types=scratch,
                     compiler_params=cp)


_sc_agg_deg = _make_sc_agg(True)
_sc_agg = _make_sc_agg(False)


BN = 400


def _linear_body(relu, p0, p1, dt, w, b, o):
    deg = jnp.maximum(jnp.sum(dt[...], axis=1, keepdims=True), 1.0)
    mean = (p0[...] + p1[...]) / deg
    out = lax.dot_general(mean, w[...], (((1,), (1,)), ((), ())),
                          preferred_element_type=jnp.float32)
    out = out + b[...]
    if relu:
        out = jnp.maximum(out, 0.0)
    o[...] = out


def _linear(featp, degt, W, b, relu):
    nb = N // BN
    return pl.pallas_call(
        functools.partial(_linear_body, relu),
        grid=(nb,),
        in_specs=[
            pl.BlockSpec((BN, D), lambda i: (i, 0)),
            pl.BlockSpec((BN, D), lambda i: (i + nb, 0)),
            pl.BlockSpec((BN, NW), lambda i: (i, 0)),
            pl.BlockSpec((D, D), lambda i: (0, 0)),
            pl.BlockSpec((1, D), lambda i: (0, 0)),
        ],
        out_specs=pl.BlockSpec((BN, D), lambda i: (i, 0)),
        out_shape=jax.ShapeDtypeStruct((N, D), jnp.float32),
    )(featp, featp, degt, W, b.reshape(1, D))


def kernel(node_features, edge_index, W1, b1, W2, b2):
    src2d = edge_index[0].astype(jnp.int32).reshape(NCH, CHUNK)
    dst2d = edge_index[1].astype(jnp.int32).reshape(NCH, CHUNK)
    znd = jnp.zeros((N, D), jnp.float32)

    featp1, degp = _sc_agg_deg(node_features, src2d, dst2d, znd)
    degt = degp.reshape(NW, N).T
    h1 = _linear(featp1, degt, W1, b1, relu=True)
    featp2 = _sc_agg(h1, src2d, dst2d, znd)
    return _linear(featp2, degt, W2, b2, relu=False)


# speedup vs baseline: 11.3875x; 1.0238x over previous
import dataclasses
import functools

import jax
import jax.numpy as jnp
from jax import lax
from jax.experimental import pallas as pl
from jax.experimental.pallas import tpu as pltpu
from jax.experimental.pallas import tpu_sc as plsc

N = 10000
E = 320000
D = 128

NC, NS = 2, 16
NW = NC * NS
CHUNK = 128
NCH = E // CHUNK
BCH = NCH // NW
XCH = NCH - BCH * NW

ZROWS = 200
NZCH = N // ZROWS
ZCH = N // 16


IBLK = 13
NBLK = BCH // IBLK


def _sc_agg_body(with_deg, table, src2d, dst2d, znd, *rest):
    if with_deg:
        (outf, outd, sidx0, sidx1, didx0, didx1, rows0, rows1, acc, dacc,
         g0, g1, h0, h1, s0, s1, i0, i1) = rest
    else:
        (outf, sidx0, sidx1, didx0, didx1, rows0, rows1, acc,
         g0, g1, h0, h1, s0, s1, i0, i1) = rest

    c = lax.axis_index("c")
    s = lax.axis_index("s")
    w = s * NC + c

    for j in range(4):
        i = s + NS * j

        @pl.when(i < NZCH)
        def _():
            r0 = i * ZROWS
            pltpu.sync_copy(znd.at[pl.ds(r0, ZROWS)], acc.at[pl.ds(r0, ZROWS)])

    cb = w * BCH
    sidxb = (sidx0, sidx1)
    didxb = (didx0, didx1)
    isem = (i0, i1)
    rowsb = (rows0, rows1)
    gsem = (g0, g1)
    gsem2 = (h0, h1)
    ssem = (s0, s1)
    ones16 = jnp.full((16,), 1.0, jnp.float32)

    pltpu.sync_copy(src2d.at[pl.ds(cb, IBLK)], sidx0)
    pltpu.sync_copy(dst2d.at[pl.ds(cb, IBLK)], didx0)

    if with_deg:
        z16 = jnp.zeros((16,), jnp.float32)

        @pl.loop(0, ZCH)
        def _(i):
            dacc[pl.ds(i * 16, 16)] = z16

    plsc.subcore_barrier()

    def issue_idx_block(m):
        q = m % 2
        pltpu.async_copy(src2d.at[pl.ds(cb + m * IBLK, IBLK)], sidxb[q],
                         isem[q])
        pltpu.async_copy(dst2d.at[pl.ds(cb + m * IBLK, IBLK)], didxb[q],
                         isem[q])

    def wait_idx_block(m):
        q = m % 2
        pltpu.make_async_copy(src2d.at[pl.ds(cb + m * IBLK, IBLK)], sidxb[q],
                              isem[q]).wait()
        pltpu.make_async_copy(dst2d.at[pl.ds(cb + m * IBLK, IBLK)], didxb[q],
                              isem[q]).wait()

    def sidx_row(k):
        m, j = divmod(k, IBLK)
        return sidxb[m % 2].at[0] if k == BCH else sidxb[m % 2].at[j]

    def didx_row(k):
        m, j = divmod(k, IBLK)
        return didxb[m % 2].at[0] if k == BCH else didxb[m % 2].at[j]

    H = CHUNK // 2

    def sidx_half(k, h):
        m, j = divmod(k, IBLK)
        j = 0 if k == BCH else j
        return sidxb[m % 2].at[j, pl.ds(h * H, H)]

    def issue_gather(k):
        pltpu.async_copy(table.at[sidx_half(k, 0)],
                         rowsb[k % 2].at[pl.ds(0, H)], gsem[k % 2])
        pltpu.async_copy(table.at[sidx_half(k, 1)],
                         rowsb[k % 2].at[pl.ds(H, H)], gsem2[k % 2])

    def wait_gather(k):
        pltpu.make_async_copy(table.at[sidx_half(k, 0)],
                              rowsb[k % 2].at[pl.ds(0, H)],
                              gsem[k % 2]).wait()
        pltpu.make_async_copy(table.at[sidx_half(k, 1)],
                              rowsb[k % 2].at[pl.ds(H, H)],
                              gsem2[k % 2]).wait()

    def start_scatter(k):
        pltpu.async_copy(rowsb[k % 2], acc.at[didx_row(k)], ssem[k % 2],
                         add=True)

    def wait_scatter(k):
        pltpu.make_async_copy(rowsb[k % 2], acc.at[didx_row(k)],
                              ssem[k % 2]).wait()

    def deg_update(k):
        if with_deg:
            m, j = divmod(k, IBLK)
            j = 0 if k == BCH else j
            for g in range(CHUNK // 16):
                plsc.addupdate_scatter(
                    dacc, [didxb[m % 2][j, pl.ds(g * 16, 16)]], ones16)

    for k in range(BCH):
        m, j = divmod(k, IBLK)
        if k == 0:
            issue_gather(0)
        wait_gather(k)
        start_scatter(k)
        if k > 0:
            wait_scatter(k - 1)
        if k < BCH - 1:
            if j == IBLK - 1:
                wait_idx_block(m + 1)
            issue_gather(k + 1)
        else:
            @pl.when(w < XCH)
            def _():
                xb = NW * BCH + w
                pltpu.make_async_copy(src2d.at[pl.ds(xb, 1)],
                                      sidxb[0].at[pl.ds(0, 1)],
                                      isem[0]).wait()
                pltpu.make_async_copy(dst2d.at[pl.ds(xb, 1)],
                                      didxb[0].at[pl.ds(0, 1)],
                                      isem[0]).wait()
                issue_gather(BCH)

        deg_update(k)
        if j == 1:
            if m + 1 < NBLK:
                issue_idx_block(m + 1)
            else:
                @pl.when(w < XCH)
                def _():
                    xb = NW * BCH + w
                    q = (m + 1) % 2
                    pltpu.async_copy(src2d.at[pl.ds(xb, 1)],
                                     sidxb[q].at[pl.ds(0, 1)], isem[q])
                    pltpu.async_copy(dst2d.at[pl.ds(xb, 1)],
                                     didxb[q].at[pl.ds(0, 1)], isem[q])

    @pl.when(w < XCH)
    def _():
        wait_gather(BCH)
        start_scatter(BCH)
        wait_scatter(BCH - 1)
        deg_update(BCH)
        wait_scatter(BCH)

    @pl.when(w >= XCH)
    def _():
        wait_scatter(BCH - 1)

    if with_deg:
        pltpu.sync_copy(dacc, outd.at[pl.ds(w * N, N)])

    plsc.subcore_barrier()

    for j in range(4):
        i = s + NS * j

        @pl.when(i < NZCH)
        def _():
            r0 = i * ZROWS
            pltpu.sync_copy(acc.at[pl.ds(r0, ZROWS)],
                            outf.at[pl.ds(c * N + r0, ZROWS)])


def _make_sc_agg(with_deg):
    mesh = plsc.VectorSubcoreMesh(core_axis_name="c", subcore_axis_name="s")
    out_type = jax.ShapeDtypeStruct((NC * N, D), jnp.float32)
    if with_deg:
        out_type = (out_type, jax.ShapeDtypeStruct((NW * N,), jnp.float32))
    scratch = [
        pltpu.VMEM((IBLK, CHUNK), jnp.int32),
        pltpu.VMEM((IBLK, CHUNK), jnp.int32),
        pltpu.VMEM((IBLK, CHUNK), jnp.int32),
        pltpu.VMEM((IBLK, CHUNK), jnp.int32),
        pltpu.VMEM((CHUNK, D), jnp.float32),
        pltpu.VMEM((CHUNK, D), jnp.float32),
        pltpu.VMEM_SHARED((N, D), jnp.float32),
    ]
    if with_deg:
        scratch.append(pltpu.VMEM((N,), jnp.float32))
    scratch += [pltpu.SemaphoreType.DMA] * 8
    cp = pltpu.CompilerParams()
    fields = pltpu.CompilerParams.__dataclass_fields__
    if "use_tc_tiling_on_sc" in fields:
        cp = dataclasses.replace(cp, use_tc_tiling_on_sc=False)
    if with_deg and "needs_layout_passes" in fields:
        cp = dataclasses.replace(cp, needs_layout_passes=False)
    return pl.kernel(functools.partial(_sc_agg_body, with_deg),
                     out_type=out_type, mesh=mesh, scratch_types=scratch,
                     compiler_params=cp)


_sc_agg_deg = _make_sc_agg(True)
_sc_agg = _make_sc_agg(False)


BN = 400


def _linear_body(relu, p0, p1, dt, w, b, o):
    deg = jnp.maximum(jnp.sum(dt[...], axis=1, keepdims=True), 1.0)
    mean = (p0[...] + p1[...]) / deg
    out = lax.dot_general(mean, w[...], (((1,), (1,)), ((), ())),
                          preferred_element_type=jnp.float32)
    out = out + b[...]
    if relu:
        out = jnp.maximum(out, 0.0)
    o[...] = out


def _linear(featp, degt, W, b, relu):
    nb = N // BN
    return pl.pallas_call(
        functools.partial(_linear_body, relu),
        grid=(nb,),
        in_specs=[
            pl.BlockSpec((BN, D), lambda i: (i, 0)),
            pl.BlockSpec((BN, D), lambda i: (i + nb, 0)),
            pl.BlockSpec((BN, NW), lambda i: (i, 0)),
            pl.BlockSpec((D, D), lambda i: (0, 0)),
            pl.BlockSpec((1, D), lambda i: (0, 0)),
        ],
        out_specs=pl.BlockSpec((BN, D), lambda i: (i, 0)),
        out_shape=jax.ShapeDtypeStruct((N, D), jnp.float32),
    )(featp, featp, degt, W, b.reshape(1, D))


def kernel(node_features, edge_index, W1, b1, W2, b2):
    src2d = edge_index[0].astype(jnp.int32).reshape(NCH, CHUNK)
    dst2d = edge_index[1].astype(jnp.int32).reshape(NCH, CHUNK)
    znd = jnp.zeros((N, D), jnp.float32)

    featp1, degp = _sc_agg_deg(node_features, src2d, dst2d, znd)
    degt = degp.reshape(NW, N).T
    h1 = _linear(featp1, degt, W1, b1, relu=True)
    featp2 = _sc_agg(h1, src2d, dst2d, znd)
    return _linear(featp2, degt, W2, b2, relu=False)


# speedup vs baseline: 12.5606x; 1.1030x over previous
import dataclasses
import functools

import jax
import jax.numpy as jnp
from jax import lax
from jax.experimental import pallas as pl
from jax.experimental.pallas import tpu as pltpu
from jax.experimental.pallas import tpu_sc as plsc

N = 10000
E = 320000
D = 128

NC, NS = 2, 16
NW = NC * NS
CHUNK = 128
NCH = E // CHUNK
BCH = NCH // NW
XCH = NCH - BCH * NW

ZROWS = 200
NZCH = N // ZROWS
ZCH = N // 16


IBLK = 13
NBLK = BCH // IBLK


def _sc_agg_body(with_deg, table, src2d, dst2d, znd, *rest):
    if with_deg:
        (outf, outd, sidx0, sidx1, didx0, didx1, rows0, rows1, acc, dacc,
         g0, g1, h0, h1, s0, s1, i0, i1) = rest
    else:
        (outf, sidx0, sidx1, didx0, didx1, rows0, rows1, acc,
         g0, g1, h0, h1, s0, s1, i0, i1) = rest

    c = lax.axis_index("c")
    s = lax.axis_index("s")
    w = s * NC + c

    for j in range(4):
        i = s + NS * j

        @pl.when(i < NZCH)
        def _():
            r0 = i * ZROWS
            pltpu.sync_copy(znd.at[pl.ds(r0, ZROWS)], acc.at[pl.ds(r0, ZROWS)])

    cb = w * BCH
    sidxb = (sidx0, sidx1)
    didxb = (didx0, didx1)
    isem = (i0, i1)
    rowsb = (rows0, rows1)
    gsem = (g0, g1)
    gsem2 = (h0, h1)
    ssem = (s0, s1)
    ones16 = jnp.full((16,), 1.0, jnp.float32)

    pltpu.sync_copy(src2d.at[pl.ds(cb, IBLK)], sidx0)
    pltpu.sync_copy(dst2d.at[pl.ds(cb, IBLK)], didx0)

    if with_deg:
        z16 = jnp.zeros((16,), jnp.float32)

        @pl.loop(0, ZCH)
        def _(i):
            dacc[pl.ds(i * 16, 16)] = z16

    plsc.subcore_barrier()

    def issue_idx_block(m):
        q = m % 2
        pltpu.async_copy(src2d.at[pl.ds(cb + m * IBLK, IBLK)], sidxb[q],
                         isem[q])
        pltpu.async_copy(dst2d.at[pl.ds(cb + m * IBLK, IBLK)], didxb[q],
                         isem[q])

    def wait_idx_block(m):
        q = m % 2
        pltpu.make_async_copy(src2d.at[pl.ds(cb + m * IBLK, IBLK)], sidxb[q],
                              isem[q]).wait()
        pltpu.make_async_copy(dst2d.at[pl.ds(cb + m * IBLK, IBLK)], didxb[q],
                              isem[q]).wait()

    def didx_row(k):
        m, j = divmod(k, IBLK)
        return didxb[m % 2].at[0] if k == BCH else didxb[m % 2].at[j]

    H = CHUNK // 2

    def sidx_half(k, h):
        m, j = divmod(k, IBLK)
        j = 0 if k == BCH else j
        return sidxb[m % 2].at[j, pl.ds(h * H, H)]

    def issue_gather(k):
        pltpu.async_copy(table.at[sidx_half(k, 0)],
                         rowsb[k % 2].at[pl.ds(0, H)], gsem[k % 2])
        pltpu.async_copy(table.at[sidx_half(k, 1)],
                         rowsb[k % 2].at[pl.ds(H, H)], gsem2[k % 2])

    def wait_gather(k):
        pltpu.make_async_copy(table.at[sidx_half(k, 0)],
                              rowsb[k % 2].at[pl.ds(0, H)],
                              gsem[k % 2]).wait()
        pltpu.make_async_copy(table.at[sidx_half(k, 1)],
                              rowsb[k % 2].at[pl.ds(H, H)],
                              gsem2[k % 2]).wait()

    def start_scatter(k):
        pltpu.async_copy(rowsb[k % 2], acc.at[didx_row(k)], ssem[k % 2],
                         add=True)

    def wait_scatter(k):
        pltpu.make_async_copy(rowsb[k % 2], acc.at[didx_row(k)],
                              ssem[k % 2]).wait()

    def deg_update(k):
        if with_deg:
            m, j = divmod(k, IBLK)
            j = 0 if k == BCH else j
            for g in range(CHUNK // 16):
                plsc.addupdate_scatter(
                    dacc, [didxb[m % 2][j, pl.ds(g * 16, 16)]], ones16)

    for k in range(BCH):
        m, j = divmod(k, IBLK)
        if k == 0:
            issue_gather(0)
        if k > 0:
            wait_scatter(k - 1)
        if k < BCH - 1:
            if j == IBLK - 1:
                wait_idx_block(m + 1)
            issue_gather(k + 1)
        else:
            @pl.when(w < XCH)
            def _():
                xb = NW * BCH + w
                pltpu.make_async_copy(src2d.at[pl.ds(xb, 1)],
                                      sidxb[0].at[pl.ds(0, 1)],
                                      isem[0]).wait()
                pltpu.make_async_copy(dst2d.at[pl.ds(xb, 1)],
                                      didxb[0].at[pl.ds(0, 1)],
                                      isem[0]).wait()
                issue_gather(BCH)

        wait_gather(k)
        start_scatter(k)
        deg_update(k)
        if j == 1:
            if m + 1 < NBLK:
                issue_idx_block(m + 1)
            else:
                @pl.when(w < XCH)
                def _():
                    xb = NW * BCH + w
                    q = (m + 1) % 2
                    pltpu.async_copy(src2d.at[pl.ds(xb, 1)],
                                     sidxb[q].at[pl.ds(0, 1)], isem[q])
                    pltpu.async_copy(dst2d.at[pl.ds(xb, 1)],
                                     didxb[q].at[pl.ds(0, 1)], isem[q])

    @pl.when(w < XCH)
    def _():
        wait_scatter(BCH - 1)
        wait_gather(BCH)
        start_scatter(BCH)
        deg_update(BCH)
        wait_scatter(BCH)

    @pl.when(w >= XCH)
    def _():
        wait_scatter(BCH - 1)

    if with_deg:
        pltpu.sync_copy(dacc, outd.at[pl.ds(w * N, N)])

    plsc.subcore_barrier()

    for j in range(4):
        i = s + NS * j

        @pl.when(i < NZCH)
        def _():
            r0 = i * ZROWS
            pltpu.sync_copy(acc.at[pl.ds(r0, ZROWS)],
                            outf.at[pl.ds(c * N + r0, ZROWS)])


def _make_sc_agg(with_deg):
    mesh = plsc.VectorSubcoreMesh(core_axis_name="c", subcore_axis_name="s")
    out_type = jax.ShapeDtypeStruct((NC * N, D), jnp.float32)
    if with_deg:
        out_type = (out_type, jax.ShapeDtypeStruct((NW * N,), jnp.float32))
    scratch = [
        pltpu.VMEM((IBLK, CHUNK), jnp.int32),
        pltpu.VMEM((IBLK, CHUNK), jnp.int32),
        pltpu.VMEM((IBLK, CHUNK), jnp.int32),
        pltpu.VMEM((IBLK, CHUNK), jnp.int32),
        pltpu.VMEM((CHUNK, D), jnp.float32),
        pltpu.VMEM((CHUNK, D), jnp.float32),
        pltpu.VMEM_SHARED((N, D), jnp.float32),
    ]
    if with_deg:
        scratch.append(pltpu.VMEM((N,), jnp.float32))
    scratch += [pltpu.SemaphoreType.DMA] * 8
    cp = pltpu.CompilerParams()
    fields = pltpu.CompilerParams.__dataclass_fields__
    if "use_tc_tiling_on_sc" in fields:
        cp = dataclasses.replace(cp, use_tc_tiling_on_sc=False)
    if with_deg and "needs_layout_passes" in fields:
        cp = dataclasses.replace(cp, needs_layout_passes=False)
    return pl.kernel(functools.partial(_sc_agg_body, with_deg),
                     out_type=out_type, mesh=mesh, scratch_types=scratch,
                     compiler_params=cp)


_sc_agg_deg = _make_sc_agg(True)
_sc_agg = _make_sc_agg(False)


BN = 400


def _linear_body(relu, p0, p1, dt, w, b, o):
    deg = jnp.maximum(jnp.sum(dt[...], axis=1, keepdims=True), 1.0)
    mean = (p0[...] + p1[...]) / deg
    out = lax.dot_general(mean, w[...], (((1,), (1,)), ((), ())),
                          preferred_element_type=jnp.float32)
    out = out + b[...]
    if relu:
        out = jnp.maximum(out, 0.0)
    o[...] = out


def _linear(featp, degt, W, b, relu):
    nb = N // BN
    return pl.pallas_call(
        functools.partial(_linear_body, relu),
        grid=(nb,),
        in_specs=[
            pl.BlockSpec((BN, D), lambda i: (i, 0)),
            pl.BlockSpec((BN, D), lambda i: (i + nb, 0)),
            pl.BlockSpec((BN, NW), lambda i: (i, 0)),
            pl.BlockSpec((D, D), lambda i: (0, 0)),
            pl.BlockSpec((1, D), lambda i: (0, 0)),
        ],
        out_specs=pl.BlockSpec((BN, D), lambda i: (i, 0)),
        out_shape=jax.ShapeDtypeStruct((N, D), jnp.float32),
    )(featp, featp, degt, W, b.reshape(1, D))


def kernel(node_features, edge_index, W1, b1, W2, b2):
    src2d = edge_index[0].astype(jnp.int32).reshape(NCH, CHUNK)
    dst2d = edge_index[1].astype(jnp.int32).reshape(NCH, CHUNK)
    znd = jnp.zeros((N, D), jnp.float32)

    featp1, degp = _sc_agg_deg(node_features, src2d, dst2d, znd)
    degt = degp.reshape(NW, N).T
    h1 = _linear(featp1, degt, W1, b1, relu=True)
    featp2 = _sc_agg(h1, src2d, dst2d, znd)
    return _linear(featp2, degt, W2, b2, relu=False)


# speedup vs baseline: 12.7313x; 1.0136x over previous
import dataclasses
import functools

import jax
import jax.numpy as jnp
from jax import lax
from jax.experimental import pallas as pl
from jax.experimental.pallas import tpu as pltpu
from jax.experimental.pallas import tpu_sc as plsc

N = 10000
E = 320000
D = 128

NC, NS = 2, 16
NW = NC * NS
CHUNK = 128
NCH = E // CHUNK
BCH = NCH // NW
XCH = NCH - BCH * NW

ZROWS = 200
NZCH = N // ZROWS
ZCH = N // 16


IBLK = 13
NBLK = BCH // IBLK


def _sc_agg_body(with_deg, table, src2d, dst2d, znd, *rest):
    if with_deg:
        (outf, outd, sidx0, sidx1, didx0, didx1, rows0, rows1, acc, dacc,
         g0, g1, h0, h1, s0, s1, i0, i1) = rest
    else:
        (outf, sidx0, sidx1, didx0, didx1, rows0, rows1, acc,
         g0, g1, h0, h1, s0, s1, i0, i1) = rest

    c = lax.axis_index("c")
    s = lax.axis_index("s")
    w = s * NC + c
    cb = w * BCH
    sidxb = (sidx0, sidx1)
    didxb = (didx0, didx1)
    isem = (i0, i1)
    rowsb = (rows0, rows1)
    gsem = (g0, g1)
    gsem2 = (h0, h1)
    ssem = (s0, s1)
    ones16 = jnp.full((16,), 1.0, jnp.float32)

    pltpu.sync_copy(src2d.at[pl.ds(cb, IBLK)], sidx0)
    pltpu.sync_copy(dst2d.at[pl.ds(cb, IBLK)], didx0)

    def issue_idx_block(m):
        q = m % 2
        pltpu.async_copy(src2d.at[pl.ds(cb + m * IBLK, IBLK)], sidxb[q],
                         isem[q])
        pltpu.async_copy(dst2d.at[pl.ds(cb + m * IBLK, IBLK)], didxb[q],
                         isem[q])

    def wait_idx_block(m):
        q = m % 2
        pltpu.make_async_copy(src2d.at[pl.ds(cb + m * IBLK, IBLK)], sidxb[q],
                              isem[q]).wait()
        pltpu.make_async_copy(dst2d.at[pl.ds(cb + m * IBLK, IBLK)], didxb[q],
                              isem[q]).wait()

    def didx_row(k):
        m, j = divmod(k, IBLK)
        return didxb[m % 2].at[0] if k == BCH else didxb[m % 2].at[j]

    H = CHUNK // 2

    def sidx_half(k, h):
        m, j = divmod(k, IBLK)
        j = 0 if k == BCH else j
        return sidxb[m % 2].at[j, pl.ds(h * H, H)]

    def issue_gather(k):
        pltpu.async_copy(table.at[sidx_half(k, 0)],
                         rowsb[k % 2].at[pl.ds(0, H)], gsem[k % 2])
        pltpu.async_copy(table.at[sidx_half(k, 1)],
                         rowsb[k % 2].at[pl.ds(H, H)], gsem2[k % 2])

    def wait_gather(k):
        pltpu.make_async_copy(table.at[sidx_half(k, 0)],
                              rowsb[k % 2].at[pl.ds(0, H)],
                              gsem[k % 2]).wait()
        pltpu.make_async_copy(table.at[sidx_half(k, 1)],
                              rowsb[k % 2].at[pl.ds(H, H)],
                              gsem2[k % 2]).wait()

    def start_scatter(k):
        pltpu.async_copy(rowsb[k % 2], acc.at[didx_row(k)], ssem[k % 2],
                         add=True)

    def wait_scatter(k):
        pltpu.make_async_copy(rowsb[k % 2], acc.at[didx_row(k)],
                              ssem[k % 2]).wait()

    def deg_update(k):
        if with_deg:
            m, j = divmod(k, IBLK)
            j = 0 if k == BCH else j
            for g in range(CHUNK // 16):
                plsc.addupdate_scatter(
                    dacc, [didxb[m % 2][j, pl.ds(g * 16, 16)]], ones16)

    issue_gather(0)

    for j in range(4):
        i = s + NS * j

        @pl.when(i < NZCH)
        def _():
            r0 = i * ZROWS
            pltpu.sync_copy(znd.at[pl.ds(r0, ZROWS)], acc.at[pl.ds(r0, ZROWS)])

    if with_deg:
        z16 = jnp.zeros((16,), jnp.float32)

        @pl.loop(0, ZCH)
        def _(i):
            dacc[pl.ds(i * 16, 16)] = z16

    plsc.subcore_barrier()

    for k in range(BCH):
        m, j = divmod(k, IBLK)
        if k > 0:
            wait_scatter(k - 1)
        if k < BCH - 1:
            if j == IBLK - 1:
                wait_idx_block(m + 1)
            issue_gather(k + 1)
        else:
            @pl.when(w < XCH)
            def _():
                xb = NW * BCH + w
                pltpu.make_async_copy(src2d.at[pl.ds(xb, 1)],
                                      sidxb[0].at[pl.ds(0, 1)],
                                      isem[0]).wait()
                pltpu.make_async_copy(dst2d.at[pl.ds(xb, 1)],
                                      didxb[0].at[pl.ds(0, 1)],
                                      isem[0]).wait()
                issue_gather(BCH)

        wait_gather(k)
        start_scatter(k)
        deg_update(k)
        if j == 1:
            if m + 1 < NBLK:
                issue_idx_block(m + 1)
            else:
                @pl.when(w < XCH)
                def _():
                    xb = NW * BCH + w
                    q = (m + 1) % 2
                    pltpu.async_copy(src2d.at[pl.ds(xb, 1)],
                                     sidxb[q].at[pl.ds(0, 1)], isem[q])
                    pltpu.async_copy(dst2d.at[pl.ds(xb, 1)],
                                     didxb[q].at[pl.ds(0, 1)], isem[q])

    @pl.when(w < XCH)
    def _():
        wait_scatter(BCH - 1)
        wait_gather(BCH)
        start_scatter(BCH)
        deg_update(BCH)
        wait_scatter(BCH)

    @pl.when(w >= XCH)
    def _():
        wait_scatter(BCH - 1)

    if with_deg:
        pltpu.sync_copy(dacc, outd.at[pl.ds(w * N, N)])

    plsc.subcore_barrier()

    for j in range(4):
        i = s + NS * j

        @pl.when(i < NZCH)
        def _():
            r0 = i * ZROWS
            pltpu.sync_copy(acc.at[pl.ds(r0, ZROWS)],
                            outf.at[pl.ds(c * N + r0, ZROWS)])


def _make_sc_agg(with_deg):
    mesh = plsc.VectorSubcoreMesh(core_axis_name="c", subcore_axis_name="s")
    out_type = jax.ShapeDtypeStruct((NC * N, D), jnp.float32)
    if with_deg:
        out_type = (out_type, jax.ShapeDtypeStruct((NW * N,), jnp.float32))
    scratch = [
        pltpu.VMEM((IBLK, CHUNK), jnp.int32),
        pltpu.VMEM((IBLK, CHUNK), jnp.int32),
        pltpu.VMEM((IBLK, CHUNK), jnp.int32),
        pltpu.VMEM((IBLK, CHUNK), jnp.int32),
        pltpu.VMEM((CHUNK, D), jnp.float32),
        pltpu.VMEM((CHUNK, D), jnp.float32),
        pltpu.VMEM_SHARED((N, D), jnp.float32),
    ]
    if with_deg:
        scratch.append(pltpu.VMEM((N,), jnp.float32))
    scratch += [pltpu.SemaphoreType.DMA] * 8
    cp = pltpu.CompilerParams()
    fields = pltpu.CompilerParams.__dataclass_fields__
    if "use_tc_tiling_on_sc" in fields:
        cp = dataclasses.replace(cp, use_tc_tiling_on_sc=False)
    if with_deg and "needs_layout_passes" in fields:
        cp = dataclasses.replace(cp, needs_layout_passes=False)
    return pl.kernel(functools.partial(_sc_agg_body, with_deg),
                     out_type=out_type, mesh=mesh, scratch_types=scratch,
                     compiler_params=cp)


_sc_agg_deg = _make_sc_agg(True)
_sc_agg = _make_sc_agg(False)


BN = 400


def _linear_body(relu, p0, p1, dt, w, b, o):
    deg = jnp.maximum(jnp.sum(dt[...], axis=1, keepdims=True), 1.0)
    mean = (p0[...] + p1[...]) / deg
    out = lax.dot_general(mean, w[...], (((1,), (1,)), ((), ())),
                          preferred_element_type=jnp.float32)
    out = out + b[...]
    if relu:
        out = jnp.maximum(out, 0.0)
    o[...] = out


def _linear(featp, degt, W, b, relu):
    nb = N // BN
    return pl.pallas_call(
        functools.partial(_linear_body, relu),
        grid=(nb,),
        in_specs=[
            pl.BlockSpec((BN, D), lambda i: (i, 0)),
            pl.BlockSpec((BN, D), lambda i: (i + nb, 0)),
            pl.BlockSpec((BN, NW), lambda i: (i, 0)),
            pl.BlockSpec((D, D), lambda i: (0, 0)),
            pl.BlockSpec((1, D), lambda i: (0, 0)),
        ],
        out_specs=pl.BlockSpec((BN, D), lambda i: (i, 0)),
        out_shape=jax.ShapeDtypeStruct((N, D), jnp.float32),
    )(featp, featp, degt, W, b.reshape(1, D))


def kernel(node_features, edge_index, W1, b1, W2, b2):
    src2d = edge_index[0].astype(jnp.int32).reshape(NCH, CHUNK)
    dst2d = edge_index[1].astype(jnp.int32).reshape(NCH, CHUNK)
    znd = jnp.zeros((N, D), jnp.float32)

    featp1, degp = _sc_agg_deg(node_features, src2d, dst2d, znd)
    degt = degp.reshape(NW, N).T
    h1 = _linear(featp1, degt, W1, b1, relu=True)
    featp2 = _sc_agg(h1, src2d, dst2d, znd)
    return _linear(featp2, degt, W2, b2, relu=False)
